# baseline (speedup 1.0000x reference)
import jax
import jax.numpy as jnp
from jax.experimental import pallas as pl
from jax.experimental.pallas import tpu as pltpu

_BATCH = 4
_CHUNKS = 16


def _copy_kernel(tbl, out, buf, in_sem, out_sem):
    num_rows = buf.shape[0]
    blk = num_rows // _CHUNKS

    def in_copy(c):
        sl = pl.ds(c * blk, blk)
        return pltpu.make_async_copy(tbl.at[sl, :], buf.at[sl, :], in_sem.at[c])

    def out_copy(b, c):
        sl = pl.ds(c * blk, blk)
        return pltpu.make_async_copy(
            buf.at[sl, :], out.at[b, sl, :], out_sem.at[b, c])

    for c in range(_CHUNKS):
        in_copy(c).start()
    for c in range(_CHUNKS):
        in_copy(c).wait()
        for b in range(_BATCH):
            out_copy(b, c).start()
    for c in range(_CHUNKS):
        for b in range(_BATCH):
            out_copy(b, c).wait()


def kernel(inputs, table):
    del inputs
    num_rows, dim = table.shape
    out = pl.pallas_call(
        _copy_kernel,
        in_specs=[pl.BlockSpec(memory_space=pl.ANY)],
        out_specs=pl.BlockSpec(memory_space=pl.ANY),
        out_shape=jax.ShapeDtypeStruct((_BATCH, num_rows, dim), table.dtype),
        scratch_shapes=[
            pltpu.VMEM((num_rows, dim), table.dtype),
            pltpu.SemaphoreType.DMA((_CHUNKS,)),
            pltpu.SemaphoreType.DMA((_BATCH, _CHUNKS)),
        ],
    )(table)
    return out


# speedup vs baseline: 1.0162x; 1.0162x over previous
import jax
import jax.numpy as jnp
from jax.experimental import pallas as pl
from jax.experimental.pallas import tpu as pltpu

_BATCH = 4
_CHUNKS = 4


def _copy_kernel(tbl, out, buf, in_sem, out_sem):
    num_rows = buf.shape[0]
    blk = num_rows // _CHUNKS

    def in_copy(c):
        sl = pl.ds(c * blk, blk)
        return pltpu.make_async_copy(tbl.at[sl, :], buf.at[sl, :], in_sem.at[c])

    def out_copy(b, c):
        sl = pl.ds(c * blk, blk)
        return pltpu.make_async_copy(
            buf.at[sl, :], out.at[b, sl, :], out_sem.at[b, c])

    for c in range(_CHUNKS):
        in_copy(c).start()
    for c in range(_CHUNKS):
        in_copy(c).wait()
        for b in range(_BATCH):
            out_copy(b, c).start()
    for c in range(_CHUNKS):
        for b in range(_BATCH):
            out_copy(b, c).wait()


def kernel(inputs, table):
    del inputs
    num_rows, dim = table.shape
    out = pl.pallas_call(
        _copy_kernel,
        in_specs=[pl.BlockSpec(memory_space=pl.ANY)],
        out_specs=pl.BlockSpec(memory_space=pl.ANY),
        out_shape=jax.ShapeDtypeStruct((_BATCH, num_rows, dim), table.dtype),
        scratch_shapes=[
            pltpu.VMEM((num_rows, dim), table.dtype),
            pltpu.SemaphoreType.DMA((_CHUNKS,)),
            pltpu.SemaphoreType.DMA((_BATCH, _CHUNKS)),
        ],
    )(table)
    return out


# speedup vs baseline: 1.0182x; 1.0020x over previous
import jax
import jax.numpy as jnp
from jax.experimental import pallas as pl
from jax.experimental.pallas import tpu as pltpu

_BATCH = 4
_CHUNKS = 2


def _copy_kernel(tbl, out, buf, in_sem, out_sem):
    num_rows = buf.shape[0]
    blk = num_rows // _CHUNKS

    def in_copy(c):
        sl = pl.ds(c * blk, blk)
        return pltpu.make_async_copy(tbl.at[sl, :], buf.at[sl, :], in_sem.at[c])

    def out_copy(b, c):
        sl = pl.ds(c * blk, blk)
        return pltpu.make_async_copy(
            buf.at[sl, :], out.at[b, sl, :], out_sem.at[b, c])

    for c in range(_CHUNKS):
        in_copy(c).start()
    for c in range(_CHUNKS):
        in_copy(c).wait()
        for b in range(_BATCH):
            out_copy(b, c).start()
    for c in range(_CHUNKS):
        for b in range(_BATCH):
            out_copy(b, c).wait()


def kernel(inputs, table):
    del inputs
    num_rows, dim = table.shape
    out = pl.pallas_call(
        _copy_kernel,
        in_specs=[pl.BlockSpec(memory_space=pl.ANY)],
        out_specs=pl.BlockSpec(memory_space=pl.ANY),
        out_shape=jax.ShapeDtypeStruct((_BATCH, num_rows, dim), table.dtype),
        scratch_shapes=[
            pltpu.VMEM((num_rows, dim), table.dtype),
            pltpu.SemaphoreType.DMA((_CHUNKS,)),
            pltpu.SemaphoreType.DMA((_BATCH, _CHUNKS)),
        ],
    )(table)
    return out
